# baseline (speedup 1.0000x reference)
import functools

import jax
import jax.numpy as jnp
from jax.experimental import pallas as pl


def _moe_block_kernel(tw_ref, tid_ref, x_ref, g_ref, u_ref, w2_ref, o_ref):
    e = pl.program_id(0)
    f = pl.program_id(1)

    @pl.when((e == 0) & (f == 0))
    def _init():
        o_ref[...] = jnp.zeros_like(o_ref)

    x = x_ref[...].astype(jnp.bfloat16)
    g = g_ref[0].astype(jnp.bfloat16)
    u = u_ref[0].astype(jnp.bfloat16)
    w2 = w2_ref[0].astype(jnp.bfloat16)
    h1g = jax.lax.dot_general(x, g, (((1,), (1,)), ((), ())),
                              preferred_element_type=jnp.float32)
    h1u = jax.lax.dot_general(x, u, (((1,), (1,)), ((), ())),
                              preferred_element_type=jnp.float32)
    a = (h1g * jax.nn.sigmoid(h1g) * h1u).astype(jnp.bfloat16)
    h2 = jax.lax.dot_general(a, w2, (((1,), (1,)), ((), ())),
                             preferred_element_type=jnp.float32)
    coef = jnp.sum(tw_ref[...] * (tid_ref[...] == e).astype(jnp.float32),
                   axis=1, keepdims=True)
    o_ref[...] += coef * h2


@functools.partial(jax.jit, static_argnames=("interpret",))
def kernel(hidden_states, w1, w2, topk_weights, topk_ids, interpret=False):
    m, d = hidden_states.shape
    e_, n2, _ = w1.shape
    dff = n2 // 2
    bff = min(dff, 256)
    nff = dff // bff

    grid = (e_, nff)
    out = pl.pallas_call(
        _moe_block_kernel,
        grid=grid,
        in_specs=[
            pl.BlockSpec((m, topk_weights.shape[1]), lambda e, f: (0, 0)),
            pl.BlockSpec((m, topk_ids.shape[1]), lambda e, f: (0, 0)),
            pl.BlockSpec((m, d), lambda e, f: (0, 0)),
            pl.BlockSpec((1, bff, d), lambda e, f: (e, f, 0)),
            pl.BlockSpec((1, bff, d), lambda e, f, _nff=nff: (e, _nff + f, 0)),
            pl.BlockSpec((1, d, bff), lambda e, f: (e, 0, f)),
        ],
        out_specs=pl.BlockSpec((m, d), lambda e, f: (0, 0)),
        out_shape=jax.ShapeDtypeStruct((m, d), jnp.float32),
        interpret=interpret,
    )(topk_weights, topk_ids, hidden_states, w1, w1, w2)
    return out


# speedup vs baseline: 1.0914x; 1.0914x over previous
import functools

import jax
import jax.numpy as jnp
from jax.experimental import pallas as pl


def _moe_block_kernel(tw_ref, tid_ref, x_ref, g_ref, u_ref, w2_ref, o_ref):
    e = pl.program_id(0)
    f = pl.program_id(1)

    x = x_ref[...].astype(jnp.bfloat16)
    g = g_ref[0].astype(jnp.bfloat16)
    u = u_ref[0].astype(jnp.bfloat16)
    w2 = w2_ref[0].astype(jnp.bfloat16)
    h1g = jax.lax.dot_general(x, g, (((1,), (1,)), ((), ())),
                              preferred_element_type=jnp.float32)
    h1u = jax.lax.dot_general(x, u, (((1,), (1,)), ((), ())),
                              preferred_element_type=jnp.float32)
    a = (h1g * jax.nn.sigmoid(h1g) * h1u).astype(jnp.bfloat16)
    h2 = jax.lax.dot_general(a, w2, (((1,), (1,)), ((), ())),
                             preferred_element_type=jnp.float32)
    coef = jnp.sum(tw_ref[...] * (tid_ref[...] == e).astype(jnp.float32),
                   axis=1, keepdims=True)
    keep = jnp.where((e == 0) & (f == 0), 0.0, 1.0)
    o_ref[...] = o_ref[...] * keep + coef * h2


@functools.partial(jax.jit, static_argnames=("interpret",))
def kernel(hidden_states, w1, w2, topk_weights, topk_ids, interpret=False):
    m, d = hidden_states.shape
    e_, n2, _ = w1.shape
    dff = n2 // 2
    bff = min(dff, 1024)
    nff = dff // bff

    grid = (e_, nff)
    out = pl.pallas_call(
        _moe_block_kernel,
        grid=grid,
        in_specs=[
            pl.BlockSpec((m, topk_weights.shape[1]), lambda e, f: (0, 0)),
            pl.BlockSpec((m, topk_ids.shape[1]), lambda e, f: (0, 0)),
            pl.BlockSpec((m, d), lambda e, f: (0, 0)),
            pl.BlockSpec((1, bff, d), lambda e, f: (e, f, 0)),
            pl.BlockSpec((1, bff, d), lambda e, f, _nff=nff: (e, _nff + f, 0)),
            pl.BlockSpec((1, d, bff), lambda e, f: (e, 0, f)),
        ],
        out_specs=pl.BlockSpec((m, d), lambda e, f: (0, 0)),
        out_shape=jax.ShapeDtypeStruct((m, d), jnp.float32),
        interpret=interpret,
    )(topk_weights, topk_ids, hidden_states, w1, w1, w2)
    return out
